# Initial kernel scaffold; baseline (speedup 1.0000x reference)
import jax
import jax.numpy as jnp
from jax.experimental import pallas as pl


def kernel(x, gate_w):
    raise NotImplementedError("write your pallas kernel here")


# speedup vs baseline: 3.5857x; 3.5857x over previous
import jax
import jax.numpy as jnp
from jax.experimental import pallas as pl

_N_EXPERTS = 8
_TOP_K = 2
_LB_WEIGHT = 0.01


def _router_body(x_ref, gw_ref, wt_ref, it_ref, facc_ref, pacc_ref, aux_ref):
    i = pl.program_id(0)
    nsteps = pl.num_programs(0)
    xb = x_ref[...]
    gw = gw_ref[...]
    l = jax.lax.dot_general(
        gw, xb, (((1,), (1,)), ((), ())),
        preferred_element_type=jnp.float32)
    blk = l.shape[1]
    e_iota = jax.lax.broadcasted_iota(jnp.int32, (_N_EXPERTS, blk), 0)

    m1 = jnp.max(l, axis=0, keepdims=True)
    i1 = jnp.min(jnp.where(l == m1, e_iota, _N_EXPERTS), axis=0, keepdims=True)
    lm = jnp.where(e_iota == i1, -jnp.inf, l)
    m2 = jnp.max(lm, axis=0, keepdims=True)
    i2 = jnp.min(jnp.where(lm == m2, e_iota, _N_EXPERTS), axis=0, keepdims=True)

    r = jnp.exp(m2 - m1)
    w1 = 1.0 / (1.0 + r)
    w2 = r / (1.0 + r)
    wt_ref[...] = jnp.concatenate([w1, w2], axis=0)
    it_ref[...] = jnp.concatenate([i1, i2], axis=0)

    t = jnp.exp(l - m1)
    denom = jnp.sum(t, axis=0, keepdims=True)
    probs = t / denom
    pc = jnp.sum(probs, axis=1, keepdims=True)
    mask = (e_iota == i1) | (e_iota == i2)
    fc = jnp.sum(jnp.where(mask, 1.0, 0.0), axis=1, keepdims=True)

    @pl.when(i == 0)
    def _init():
        facc_ref[...] = jnp.zeros_like(facc_ref)
        pacc_ref[...] = jnp.zeros_like(pacc_ref)

    facc_ref[...] += jnp.broadcast_to(fc, facc_ref.shape)
    pacc_ref[...] += jnp.broadcast_to(pc, pacc_ref.shape)

    @pl.when(i == nsteps - 1)
    def _fin():
        f = facc_ref[:, 0:1]
        p = pacc_ref[:, 0:1]
        s = jnp.sum(f * p)
        n_tok = jnp.float32(nsteps * blk)
        aux_ref[...] = (_N_EXPERTS * _LB_WEIGHT * s / (n_tok * n_tok)).reshape(1, 1)


def kernel(x, gate_w):
    b, s, d = x.shape
    n_tok = b * s
    xf = x.reshape(n_tok, d)
    blk = 512
    grid = n_tok // blk

    out_shapes = (
        jax.ShapeDtypeStruct((_TOP_K, n_tok), jnp.float32),
        jax.ShapeDtypeStruct((_TOP_K, n_tok), jnp.int32),
        jax.ShapeDtypeStruct((_N_EXPERTS, 128), jnp.float32),
        jax.ShapeDtypeStruct((_N_EXPERTS, 128), jnp.float32),
        jax.ShapeDtypeStruct((1, 1), jnp.float32),
    )
    wt, it, _, _, aux = pl.pallas_call(
        _router_body,
        grid=(grid,),
        in_specs=[
            pl.BlockSpec((blk, d), lambda i: (i, 0)),
            pl.BlockSpec((_N_EXPERTS, d), lambda i: (0, 0)),
        ],
        out_specs=[
            pl.BlockSpec((_TOP_K, blk), lambda i: (0, i)),
            pl.BlockSpec((_TOP_K, blk), lambda i: (0, i)),
            pl.BlockSpec((_N_EXPERTS, 128), lambda i: (0, 0)),
            pl.BlockSpec((_N_EXPERTS, 128), lambda i: (0, 0)),
            pl.BlockSpec((1, 1), lambda i: (0, 0)),
        ],
        out_shape=out_shapes,
    )(xf, gate_w)

    top_k_weights = wt.T.reshape(b, s, _TOP_K)
    top_k_indices = it.T.reshape(b, s, _TOP_K)
    return (top_k_weights, top_k_indices, aux[0, 0])


# speedup vs baseline: 4.4036x; 1.2281x over previous
import jax
import jax.numpy as jnp
from jax.experimental import pallas as pl

_N_EXPERTS = 8
_TOP_K = 2
_LB_WEIGHT = 0.01


def _router_body(x_ref, gw_ref, wt_ref, it_ref, facc_ref, pacc_ref, aux_ref):
    i = pl.program_id(0)
    nsteps = pl.num_programs(0)
    xb = x_ref[...]
    gw = gw_ref[...]
    l = jax.lax.dot_general(
        gw, xb, (((1,), (1,)), ((), ())),
        preferred_element_type=jnp.float32)
    blk = l.shape[1]
    e_iota = jax.lax.broadcasted_iota(jnp.int32, (_N_EXPERTS, blk), 0)

    m1 = jnp.max(l, axis=0, keepdims=True)
    i1 = jnp.min(jnp.where(l == m1, e_iota, _N_EXPERTS), axis=0, keepdims=True)
    lm = jnp.where(e_iota == i1, -jnp.inf, l)
    m2 = jnp.max(lm, axis=0, keepdims=True)
    i2 = jnp.min(jnp.where(lm == m2, e_iota, _N_EXPERTS), axis=0, keepdims=True)

    r = jnp.exp(m2 - m1)
    w1 = 1.0 / (1.0 + r)
    w2 = r / (1.0 + r)
    wt_ref[...] = jnp.concatenate([w1, w2], axis=0)
    it_ref[...] = jnp.concatenate([i1, i2], axis=0)

    t = jnp.exp(l - m1)
    denom = jnp.sum(t, axis=0, keepdims=True)
    probs = t / denom
    pc = jnp.sum(probs, axis=1, keepdims=True)
    mask = (e_iota == i1) | (e_iota == i2)
    fc = jnp.sum(jnp.where(mask, 1.0, 0.0), axis=1, keepdims=True)

    @pl.when(i == 0)
    def _init():
        facc_ref[...] = jnp.zeros_like(facc_ref)
        pacc_ref[...] = jnp.zeros_like(pacc_ref)

    facc_ref[...] += jnp.broadcast_to(fc, facc_ref.shape)
    pacc_ref[...] += jnp.broadcast_to(pc, pacc_ref.shape)

    @pl.when(i == nsteps - 1)
    def _fin():
        f = facc_ref[:, 0:1]
        p = pacc_ref[:, 0:1]
        s = jnp.sum(f * p)
        n_tok = jnp.float32(nsteps * blk)
        aux_ref[...] = (_N_EXPERTS * _LB_WEIGHT * s / (n_tok * n_tok)).reshape(1, 1)


def kernel(x, gate_w):
    b, s, d = x.shape
    n_tok = b * s
    xf = x.reshape(n_tok, d)
    blk = 1024
    grid = n_tok // blk

    out_shapes = (
        jax.ShapeDtypeStruct((_TOP_K, n_tok), jnp.float32),
        jax.ShapeDtypeStruct((_TOP_K, n_tok), jnp.int32),
        jax.ShapeDtypeStruct((_N_EXPERTS, 128), jnp.float32),
        jax.ShapeDtypeStruct((_N_EXPERTS, 128), jnp.float32),
        jax.ShapeDtypeStruct((1, 1), jnp.float32),
    )
    wt, it, _, _, aux = pl.pallas_call(
        _router_body,
        grid=(grid,),
        in_specs=[
            pl.BlockSpec((blk, d), lambda i: (i, 0)),
            pl.BlockSpec((_N_EXPERTS, d), lambda i: (0, 0)),
        ],
        out_specs=[
            pl.BlockSpec((_TOP_K, blk), lambda i: (0, i)),
            pl.BlockSpec((_TOP_K, blk), lambda i: (0, i)),
            pl.BlockSpec((_N_EXPERTS, 128), lambda i: (0, 0)),
            pl.BlockSpec((_N_EXPERTS, 128), lambda i: (0, 0)),
            pl.BlockSpec((1, 1), lambda i: (0, 0)),
        ],
        out_shape=out_shapes,
    )(xf, gate_w)

    top_k_weights = wt.T.reshape(b, s, _TOP_K)
    top_k_indices = it.T.reshape(b, s, _TOP_K)
    return (top_k_weights, top_k_indices, aux[0, 0])
